# Initial kernel scaffold; baseline (speedup 1.0000x reference)
import jax
import jax.numpy as jnp
from jax.experimental import pallas as pl


def kernel(x, embed_weight):
    raise NotImplementedError("write your pallas kernel here")


# speedup vs baseline: 1.8963x; 1.8963x over previous
import functools

import jax
import jax.numpy as jnp
from jax import lax
from jax.experimental import pallas as pl
from jax.experimental.pallas import tpu as pltpu
from jax.experimental.pallas import tpu_sc as plsc

VOCAB = 8192
D = 8192
NC, NS = 2, 16
NW = NC * NS
BTOT = 16 * 256
BPW = BTOT // NW
K = 4
NCH = BPW // K


def _gather_body(idx_hbm, tbl_hbm, out_hbm, idx_v, buf0, buf1, sem0, sem1):
    wid = lax.axis_index("s") * NC + lax.axis_index("c")
    base = wid * BPW
    pltpu.sync_copy(idx_hbm.at[wid], idx_v)
    bufs = (buf0, buf1)
    sems = (sem0, sem1)

    for b in range(2):
        pltpu.async_copy(tbl_hbm.at[idx_v.at[b]], bufs[b], sems[b])

    def pair_body(i, carry):
        c0 = 2 * i
        for b in range(2):
            c = c0 + b
            pltpu.make_async_copy(tbl_hbm.at[idx_v.at[c]], bufs[b], sems[b]).wait()
            pltpu.sync_copy(bufs[b], out_hbm.at[pl.ds(base + c * K, K)])
            pltpu.async_copy(tbl_hbm.at[idx_v.at[c + 2]], bufs[b], sems[b])
        return carry

    lax.fori_loop(0, NCH // 2 - 1, pair_body, 0)

    for b in range(2):
        c = NCH - 2 + b
        pltpu.make_async_copy(tbl_hbm.at[idx_v.at[c]], bufs[b], sems[b]).wait()
        pltpu.sync_copy(bufs[b], out_hbm.at[pl.ds(base + c * K, K)])


_sc_gather = functools.partial(
    pl.kernel,
    mesh=plsc.VectorSubcoreMesh(core_axis_name="c", subcore_axis_name="s"),
    out_type=jax.ShapeDtypeStruct((BTOT, D), jnp.float32),
    scratch_types=[
        pltpu.VMEM((NCH, K), jnp.int32),
        pltpu.VMEM((K, D), jnp.float32),
        pltpu.VMEM((K, D), jnp.float32),
        pltpu.SemaphoreType.DMA,
        pltpu.SemaphoreType.DMA,
    ],
)(_gather_body)


def kernel(x, embed_weight):
    B, L = x.shape
    idx = x.reshape(NW, NCH, K).astype(jnp.int32)
    out = _sc_gather(idx, embed_weight)
    return out.reshape(B, L, D)
